# Initial kernel scaffold; baseline (speedup 1.0000x reference)
import jax
import jax.numpy as jnp
from jax.experimental import pallas as pl


def kernel(x, edge_index, batch, W0, b0, W1, b1, W2, b2, Wf, bf):
    raise NotImplementedError("write your pallas kernel here")


# speedup vs baseline: 11.1786x; 11.1786x over previous
import functools

import jax
import jax.numpy as jnp
from jax import lax
from jax.experimental import pallas as pl
from jax.experimental.pallas import tpu as pltpu
from jax.experimental.pallas import tpu_sc as plsc

N = 10000
E = 320000
D_IN = 128
H = 128
C = 10
G = 128

NC = 2
NS = 16
NW = NC * NS
EDGES_PER_TILE = E // NW
CHUNK = 80
NUM_CHUNKS = EDGES_PER_TILE // CHUNK

ROWS_PER_TILE = 624
REM_OFF = ROWS_PER_TILE * NS
REM_LEN = N - REM_OFF

_sc_mesh = plsc.VectorSubcoreMesh(core_axis_name="c", subcore_axis_name="s")


@functools.partial(
    pl.kernel,
    out_type=jax.ShapeDtypeStruct((NC * N,), jnp.float32),
    mesh=_sc_mesh,
    scratch_types=[
        pltpu.VMEM((CHUNK,), jnp.int32),
        pltpu.VMEM((CHUNK,), jnp.float32),
        pltpu.VMEM((ROWS_PER_TILE,), jnp.float32),
        pltpu.VMEM_SHARED((N,), jnp.float32),
    ],
)
def _deg_kernel(dst_hbm, out_hbm, idx_v, ones_v, zero_v, deg_sh):
    c = lax.axis_index("c")
    s = lax.axis_index("s")
    w = s * NC + c

    for j in range(CHUNK // 16):
        ones_v[pl.ds(j * 16, 16)] = jnp.ones((16,), jnp.float32)
    for j in range(ROWS_PER_TILE // 16):
        zero_v[pl.ds(j * 16, 16)] = jnp.zeros((16,), jnp.float32)

    if True:
        r0 = s * ROWS_PER_TILE
        pltpu.sync_copy(zero_v, deg_sh.at[pl.ds(r0, ROWS_PER_TILE)])

        @pl.when(s == 0)
        def _():
            pltpu.sync_copy(zero_v.at[pl.ds(0, REM_LEN)],
                            deg_sh.at[pl.ds(REM_OFF, REM_LEN)])

        plsc.subcore_barrier()

        base = w * EDGES_PER_TILE

        def body(i, carry):
            off = base + i * CHUNK
            pltpu.sync_copy(dst_hbm.at[pl.ds(off, CHUNK)], idx_v)
            pltpu.sync_copy(ones_v, deg_sh.at[idx_v], add=True)
            return carry

        lax.fori_loop(0, NUM_CHUNKS, body, 0)
        plsc.subcore_barrier()

        pltpu.sync_copy(deg_sh.at[pl.ds(r0, ROWS_PER_TILE)], zero_v)
        pltpu.sync_copy(zero_v, out_hbm.at[pl.ds(c * N + r0, ROWS_PER_TILE)])

        @pl.when(s == 0)
        def _():
            pltpu.sync_copy(deg_sh.at[pl.ds(REM_OFF, REM_LEN)],
                            zero_v.at[pl.ds(0, REM_LEN)])
            pltpu.sync_copy(zero_v.at[pl.ds(0, REM_LEN)],
                            out_hbm.at[pl.ds(c * N + REM_OFF, REM_LEN)])


@functools.partial(
    pl.kernel,
    out_type=jax.ShapeDtypeStruct((NC * N, H), jnp.float32),
    mesh=_sc_mesh,
    scratch_types=[
        pltpu.VMEM((CHUNK,), jnp.int32),
        pltpu.VMEM((CHUNK,), jnp.int32),
        pltpu.VMEM((CHUNK, H), jnp.float32),
        pltpu.VMEM_SHARED((N, H), jnp.float32),
        pltpu.SemaphoreType.DMA,
    ],
)
def _gather_scatter_kernel(hp_hbm, src_hbm, dst_hbm, zeros_hbm, out_hbm,
                           si_v, di_v, rows_v, agg_sh, sem):
    c = lax.axis_index("c")
    s = lax.axis_index("s")
    w = s * NC + c

    if True:
        r0 = s * ROWS_PER_TILE

        @pl.when(c == 0)
        def _():
            pltpu.sync_copy(hp_hbm.at[pl.ds(r0, ROWS_PER_TILE)],
                            agg_sh.at[pl.ds(r0, ROWS_PER_TILE)])

            @pl.when(s == 0)
            def _():
                pltpu.sync_copy(hp_hbm.at[pl.ds(REM_OFF, REM_LEN)],
                                agg_sh.at[pl.ds(REM_OFF, REM_LEN)])

        @pl.when(c == 1)
        def _():
            pltpu.sync_copy(zeros_hbm.at[pl.ds(r0, ROWS_PER_TILE)],
                            agg_sh.at[pl.ds(r0, ROWS_PER_TILE)])

            @pl.when(s == 0)
            def _():
                pltpu.sync_copy(zeros_hbm.at[pl.ds(REM_OFF, REM_LEN)],
                                agg_sh.at[pl.ds(REM_OFF, REM_LEN)])

        plsc.subcore_barrier()

        base = w * EDGES_PER_TILE

        def body(i, carry):
            off = base + i * CHUNK
            pltpu.sync_copy(src_hbm.at[pl.ds(off, CHUNK)], si_v)
            pltpu.sync_copy(dst_hbm.at[pl.ds(off, CHUNK)], di_v)
            pltpu.async_copy(hp_hbm.at[si_v], rows_v, sem).wait()
            pltpu.sync_copy(rows_v, agg_sh.at[di_v], add=True)
            return carry

        lax.fori_loop(0, NUM_CHUNKS, body, 0)
        plsc.subcore_barrier()

        pltpu.sync_copy(agg_sh.at[pl.ds(r0, ROWS_PER_TILE)],
                        out_hbm.at[pl.ds(c * N + r0, ROWS_PER_TILE)])

        @pl.when(s == 0)
        def _():
            pltpu.sync_copy(agg_sh.at[pl.ds(REM_OFF, REM_LEN)],
                            out_hbm.at[pl.ds(c * N + REM_OFF, REM_LEN)])


def _tc1_body(x_ref, w_ref, deg0_ref, deg1_ref, hp_ref, dis_ref):
    deg = deg0_ref[...] + deg1_ref[...] + 1.0
    dis = lax.rsqrt(jnp.maximum(deg, 1.0))
    dis_ref[...] = dis
    h = jnp.dot(x_ref[...], w_ref[...], preferred_element_type=jnp.float32)
    hp_ref[...] = h * dis


def _tc_mid_body(p0_ref, p1_ref, dis_ref, b_ref, w_ref, hp_ref):
    dis = dis_ref[...]
    h = jnp.maximum((p0_ref[...] + p1_ref[...]) * dis + b_ref[...], 0.0)
    hp_ref[...] = jnp.dot(h, w_ref[...],
                          preferred_element_type=jnp.float32) * dis


def _tc_final_body(p0_ref, p1_ref, dis_ref, b_ref, batch_ref, wf_ref, bf_ref,
                   out_ref):
    h = jnp.maximum((p0_ref[...] + p1_ref[...]) * dis_ref[...] + b_ref[...],
                    0.0)
    groups = lax.broadcasted_iota(jnp.int32, (1, G), 1)
    onehot = (batch_ref[...] == groups).astype(jnp.float32)
    dn = (((0,), (0,)), ((), ()))
    sums = lax.dot_general(onehot, h, dn, preferred_element_type=jnp.float32)
    ones_col = jnp.ones((N, 1), jnp.float32)
    cnt = lax.dot_general(onehot, ones_col, dn,
                          preferred_element_type=jnp.float32)
    pooled = sums / jnp.maximum(cnt, 1.0)
    out_ref[...] = jnp.dot(pooled, wf_ref[...],
                           preferred_element_type=jnp.float32) + bf_ref[...]


def kernel(x, edge_index, batch, W0, b0, W1, b1, W2, b2, Wf, bf):
    src = edge_index[0]
    dst = edge_index[1]
    zeros = jnp.zeros((N, H), jnp.float32)

    deg_flat = _deg_kernel(dst)
    deg0 = deg_flat[:N].reshape(N, 1)
    deg1 = deg_flat[N:].reshape(N, 1)

    tc1 = pl.pallas_call(
        _tc1_body,
        out_shape=(jax.ShapeDtypeStruct((N, H), jnp.float32),
                   jax.ShapeDtypeStruct((N, 1), jnp.float32)),
    )
    hp, dis = tc1(x, W0, deg0, deg1)

    tc_mid = pl.pallas_call(
        _tc_mid_body,
        out_shape=jax.ShapeDtypeStruct((N, H), jnp.float32),
    )

    for (bias, w_next) in ((b0, W1), (b1, W2)):
        part = _gather_scatter_kernel(hp, src, dst, zeros)
        hp = tc_mid(part[:N], part[N:], dis, bias.reshape(1, H), w_next)

    part = _gather_scatter_kernel(hp, src, dst, zeros)

    tc_final = pl.pallas_call(
        _tc_final_body,
        out_shape=jax.ShapeDtypeStruct((G, C), jnp.float32),
    )
    out = tc_final(part[:N], part[N:], dis, b2.reshape(1, H),
                   batch.reshape(N, 1), Wf, bf.reshape(1, C))
    return out
